# Initial kernel scaffold; baseline (speedup 1.0000x reference)
import jax
import jax.numpy as jnp
from jax.experimental import pallas as pl


def kernel(x, values, skip_w, skip_b):
    raise NotImplementedError("write your pallas kernel here")


# speedup vs baseline: 10.8976x; 10.8976x over previous
import functools

import jax
import jax.numpy as jnp
from jax.experimental import pallas as pl
from jax.experimental.pallas import tpu as pltpu


def _tc_body(x_ref, vt_ref, swt_ref, sb_ref, out_ref, *, K):
    bm, D = x_ref.shape
    xc = jnp.clip(x_ref[:], -1.0, 1.0)
    t = (xc + 1.0) * ((K - 1) * 0.5)
    lf = jnp.clip(jnp.floor(t), 0.0, K - 2.0)
    w = t - lf
    li = lf.astype(jnp.int32)
    kk = jax.lax.broadcasted_iota(jnp.int32, (bm, D, K), 2)
    l3 = li[:, :, None]
    w3 = w[:, :, None]
    W = jnp.where(kk == l3, 1.0 - w3, jnp.where(kk == l3 + 1, w3, 0.0))
    Wf = W.reshape(bm, D * K)
    y = jnp.dot(Wf, vt_ref[:], preferred_element_type=jnp.float32)
    y = y + jnp.dot(xc, swt_ref[:], preferred_element_type=jnp.float32)
    out_ref[:] = y + sb_ref[:]


def kernel(x, values, skip_w, skip_b):
    B, D = x.shape
    O, _, K = values.shape
    vt = values.reshape(O, D * K).T
    swt = skip_w.T
    sb2 = skip_b[None, :]
    bm = min(B, 256)
    grid = (B // bm,)
    return pl.pallas_call(
        functools.partial(_tc_body, K=K),
        grid=grid,
        in_specs=[
            pl.BlockSpec((bm, D), lambda i: (i, 0)),
            pl.BlockSpec((D * K, O), lambda i: (0, 0)),
            pl.BlockSpec((D, O), lambda i: (0, 0)),
            pl.BlockSpec((1, O), lambda i: (0, 0)),
        ],
        out_specs=pl.BlockSpec((bm, O), lambda i: (i, 0)),
        out_shape=jax.ShapeDtypeStruct((B, O), jnp.float32),
    )(x, vt, swt, sb2)
